# baseline (speedup 1.0000x reference)
import jax
import jax.numpy as jnp
from jax.experimental import pallas as pl
from jax.experimental.pallas import tpu as pltpu

SEQ_OUT = 49
BATCH = 1024
VOCAB = 1000
EMB_ROWS = 100
EMB_DIM = 10


def _bcast_kernel(idx_ref, emb_ref, w_ref, b_ref, out_ref, acc_ref):
    step = pl.program_id(0)

    @pl.when(step == 0)
    def _compute():
        idx = idx_ref[0]
        rows = jax.lax.broadcasted_iota(jnp.int32, (EMB_ROWS, BATCH), 0)
        onehot = (rows == idx).astype(jnp.float32)
        pooled = jax.lax.dot_general(
            onehot, emb_ref[:, :],
            dimension_numbers=(((0,), (0,)), ((), ())),
            preferred_element_type=jnp.float32,
        )
        out = jax.lax.dot_general(
            pooled, w_ref[:, :],
            dimension_numbers=(((1,), (0,)), ((), ())),
            preferred_element_type=jnp.float32,
        )
        acc_ref[:, :] = out + b_ref[:, :]

    out_ref[0] = acc_ref[:, :]


def kernel(question, answer, emb_table, lin_w, lin_b):
    del question
    idx = answer[:1].reshape(1, 1, BATCH).astype(jnp.int32)
    w_t = lin_w.T
    b2 = lin_b.reshape(1, VOCAB)

    out = pl.pallas_call(
        _bcast_kernel,
        grid=(SEQ_OUT,),
        in_specs=[
            pl.BlockSpec((1, 1, BATCH), lambda i: (0, 0, 0)),
            pl.BlockSpec((EMB_ROWS, EMB_DIM), lambda i: (0, 0)),
            pl.BlockSpec((EMB_DIM, VOCAB), lambda i: (0, 0)),
            pl.BlockSpec((1, VOCAB), lambda i: (0, 0)),
        ],
        out_specs=pl.BlockSpec((1, BATCH, VOCAB), lambda i: (i, 0, 0)),
        out_shape=jax.ShapeDtypeStruct((SEQ_OUT, BATCH, VOCAB), jnp.float32),
        scratch_shapes=[pltpu.VMEM((BATCH, VOCAB), jnp.float32)],
    )(idx, emb_table, w_t, b2)
    return out
